# baseline (speedup 1.0000x reference)
import functools
import math

import jax
import jax.numpy as jnp
from jax import lax
from jax.experimental import pallas as pl
from jax.experimental.pallas import tpu as pltpu
from jax.experimental.pallas import tpu_sc as plsc

_N = 10000
_R = 100
_E = 160000
_D = 128
_RATIO = 0.5

_NC = 2
_NS = 16
_EPT = _E // _NS
_CH = 80
_NCHUNK = _EPT // _CH
_ACC_ROWS = 10240


def _edge_body(table_hbm, gidx_hbm, dstc_hbm, zeros_hbm, out_hbm,
               gall, dall, sb0, sb1, rows0, rows1, acc, sem0, sem1):
    c = lax.axis_index("c")
    s = lax.axis_index("s")
    lo = c * _N
    hi = lo + _N
    base_e = s * _EPT

    pltpu.sync_copy(zeros_hbm.at[pl.ds(0, _ACC_ROWS // _NS)],
                    acc.at[pl.ds(s * (_ACC_ROWS // _NS), _ACC_ROWS // _NS)])
    pltpu.sync_copy(gidx_hbm.at[pl.ds(base_e, _EPT)], gall)
    pltpu.sync_copy(dstc_hbm.at[pl.ds(base_e, _EPT)], dall)
    plsc.subcore_barrier()

    trash = jnp.arange(16, dtype=jnp.int32) + _N

    def fire(ci, rbuf, sem):
        pltpu.async_copy(table_hbm.at[gall.at[pl.ds(ci * _CH, _CH)]],
                         rbuf, sem)

    def drain(rbuf, sem):
        pltpu.make_async_copy(zeros_hbm.at[pl.ds(0, _CH)], rbuf, sem).wait()

    def process(ci, rbuf, sbuf):
        for j in range(_CH // 16):
            dv = dall[pl.ds(ci * _CH + j * 16, 16)]
            mine = (dv >= lo) & (dv < hi)
            sbuf[pl.ds(j * 16, 16)] = jnp.where(mine, dv - lo, trash)
        pltpu.sync_copy(rbuf, acc.at[sbuf], add=True)

    fire(0, rows0, sem0)

    def lbody(i, carry):
        fire(2 * i + 1, rows1, sem1)
        drain(rows0, sem0)
        process(2 * i, rows0, sb0)
        fire(jnp.minimum(2 * i + 2, _NCHUNK - 1), rows0, sem0)
        drain(rows1, sem1)
        process(2 * i + 1, rows1, sb1)
        return carry

    lax.fori_loop(0, (_NCHUNK - 1) // 2, lbody, 0)
    drain(rows0, sem0)
    process(_NCHUNK - 1, rows0, sb0)
    plsc.subcore_barrier()

    @pl.when(s == 0)
    def _():
        pltpu.sync_copy(acc.at[pl.ds(0, _N)], out_hbm.at[pl.ds(lo, _N)])


def _sc_edge_pass(table, gidx, dstc):
    mesh = plsc.VectorSubcoreMesh(core_axis_name="c", subcore_axis_name="s",
                                  num_cores=_NC, num_subcores=_NS)
    f = pl.kernel(
        _edge_body,
        out_type=jax.ShapeDtypeStruct((2 * _N, _D), jnp.float32),
        mesh=mesh,
        scratch_types=[
            pltpu.VMEM((_EPT,), jnp.int32),
            pltpu.VMEM((_EPT,), jnp.int32),
            pltpu.VMEM((_CH,), jnp.int32),
            pltpu.VMEM((_CH,), jnp.int32),
            pltpu.VMEM((_CH, _D), jnp.float32),
            pltpu.VMEM((_CH, _D), jnp.float32),
            pltpu.VMEM_SHARED((_ACC_ROWS, _D), jnp.float32),
            pltpu.SemaphoreType.DMA,
            pltpu.SemaphoreType.DMA,
        ],
    )
    zeros = jnp.zeros((_ACC_ROWS // _NS, _D), jnp.float32)
    return f(table, gidx, dstc, zeros)


def _mm_body(x_ref, w_ref, o_ref):
    o_ref[...] = jnp.dot(x_ref[...], w_ref[...],
                         preferred_element_type=jnp.float32,
                         precision=jax.lax.Precision.HIGHEST)


def _mm(x, w, bm=2000):
    m, k = x.shape
    j = w.shape[1]
    if m % bm != 0:
        return pl.pallas_call(
            _mm_body,
            out_shape=jax.ShapeDtypeStruct((m, j), jnp.float32),
        )(x, w)
    return pl.pallas_call(
        _mm_body,
        grid=(m // bm,),
        in_specs=[pl.BlockSpec((bm, k), lambda i: (i, 0)),
                  pl.BlockSpec((k, j), lambda i: (0, 0))],
        out_specs=pl.BlockSpec((bm, j), lambda i: (i, 0)),
        out_shape=jax.ShapeDtypeStruct((m, j), jnp.float32),
    )(x, w)


def _fast_epilogue_body(ai_ref, ao_ref, p2_ref, dd_ref, lm_ref, b_ref,
                        alive_ref, o_ref):
    agg = (ai_ref[...] - p2_ref[:, :_D]) * dd_ref[:, 0:1] \
        + (ao_ref[...] - p2_ref[:, _D:]) * dd_ref[:, 1:2]
    val = jnp.tanh((agg + lm_ref[...]) / 3.0 + b_ref[...])
    o_ref[...] = val * alive_ref[...]


def _fast_epilogue(agg_in, agg_out, p2, di_d_in, di_d_out, loop_msg, bias,
                   alive):
    n, d = agg_in.shape
    dd = jnp.stack([di_d_in, di_d_out], axis=1)
    return pl.pallas_call(
        _fast_epilogue_body,
        out_shape=jax.ShapeDtypeStruct((n, d), jnp.float32),
    )(agg_in, agg_out, p2, dd, loop_msg,
      jnp.broadcast_to(bias[None, :], (n, d)),
      jnp.broadcast_to(alive[:, None].astype(jnp.float32), (n, d)))


def _di(deg):
    safe = jnp.where(deg > 0, deg, 1.0)
    return jnp.where(deg > 0, safe ** -0.5, 0.0)


def _level_degrees(idx_cat, dir_in, valid):
    w2 = jnp.stack([dir_in & valid, (~dir_in) & valid],
                   axis=1).astype(jnp.float32)
    w4 = jnp.concatenate([w2, w2], axis=0)
    deg = jnp.zeros((2 * _N, 2), jnp.float32).at[idx_cat].add(w4)
    di_d = _di(deg[:_N])
    di_s = _di(deg[_N:])
    di_s_cat = jnp.concatenate([di_s[:, 0], di_s[:, 1]])
    di_d_cat = jnp.concatenate([di_d[:, 0], di_d[:, 1]])
    return di_s_cat, di_d_cat, di_d[:, 0], di_d[:, 1]


def _conv_exact(x, rel_t, src, dst, et, dir_in, off, di_s_cat, di_d_cat,
                valid, W_in, W_out, W_loop, W_rel, loop_rel, bias, alive):
    comp = jnp.take(x, src, axis=0) - jnp.take(rel_t, et, axis=0)
    di_prod = jnp.take(di_s_cat, src + off) * jnp.take(di_d_cat, dst + off)
    ni = jnp.where(dir_in & valid, di_prod, 0.0)
    no = jnp.where((~dir_in) & valid, di_prod, 0.0)
    msg = (comp @ W_in) * ni[:, None] + (comp @ W_out) * no[:, None]
    agg = jnp.zeros((_N, _D), x.dtype).at[dst].add(msg)
    loop_msg = (x - loop_rel[None, :]) @ W_loop
    out = jnp.tanh((agg + loop_msg) / 3.0 + bias)
    return out * alive[:, None], rel_t @ W_rel


def _conv_fast(x, alive, gidx, dst_cat, S, di_s_cat, di_d_in,
               di_d_out, rel_t, W_in, W_out, W_loop, W_rel, loop_rel,
               bias):
    y = _mm(x, jnp.concatenate([W_in, W_out, W_loop], axis=1))
    xw_in, xw_out, xw_loop = y[:, :_D], y[:, _D:2 * _D], y[:, 2 * _D:]
    rw = _mm(rel_t, jnp.concatenate([W_in, W_out, W_rel], axis=1))
    rw_in, rw_out, rel_new = rw[:, :_D], rw[:, _D:2 * _D], rw[:, 2 * _D:]

    di_s_in, di_s_out = di_s_cat[:_N], di_s_cat[_N:]
    table = jnp.concatenate([
        xw_in * di_s_in[:, None],
        xw_out * di_s_out[:, None],
    ], axis=0)

    agg_cat = _sc_edge_pass(table, gidx, dst_cat)

    rw_blk = jnp.block([[rw_in, jnp.zeros((2 * _R, _D), jnp.float32)],
                        [jnp.zeros((2 * _R, _D), jnp.float32), rw_out]])
    p2 = _mm(S, rw_blk)

    loop_msg = xw_loop - (loop_rel @ W_loop)[None, :]
    x_new = _fast_epilogue(agg_cat[:_N], agg_cat[_N:], p2, di_d_in, di_d_out,
                           loop_msg, bias, alive)
    return x_new, rel_new


def _pool_exact(x, alive, p, k):
    score = jnp.tanh((x @ p) / jnp.linalg.norm(p))
    keys = jnp.where(alive, score, -jnp.inf)
    _, perm = jax.lax.top_k(keys, k)
    alive_new = jnp.zeros((_N,), bool).at[perm].set(True)
    x_new = x * score[:, None] * alive_new[:, None].astype(x.dtype)
    return x_new, alive_new


def kernel(sub, rel, edge_index, edge_type, init_embed, init_rel, W_in,
           W_out, W_loop, W_rel, loop_rel, conv_bias, pool_p):
    src, dst = edge_index[0], edge_index[1]
    et = edge_type

    ones_n = jnp.ones((_N,), bool)
    ones_e = jnp.ones((_E,), bool)

    k1 = int(math.ceil(_RATIO * _N))
    k2 = int(math.ceil(_RATIO * k1))

    dir_in = et < _R
    off = jnp.where(dir_in, 0, _N).astype(src.dtype)
    setp = (et + jnp.where(dir_in, 0, 2 * _R)).astype(src.dtype)

    idx_cat = jnp.concatenate([dst, src + _N])

    ds0, dd0, dd0_in, dd0_out = _level_degrees(idx_cat, dir_in, ones_e)
    x0, r0 = _conv_exact(init_embed, init_rel, src, dst, et, dir_in, off,
                         ds0, dd0, ones_e, W_in[0], W_out[0], W_loop[0],
                         W_rel[0], loop_rel[0], conv_bias[0], ones_n)

    xp1, alive1 = _pool_exact(x0, ones_n, pool_p[0], k1)
    v1 = jnp.take(alive1, src) & jnp.take(alive1, dst)
    ds1, dd1, dd1_in, dd1_out = _level_degrees(idx_cat, dir_in, v1)
    x1, r1 = _conv_exact(xp1, r0, src, dst, et, dir_in, off,
                         ds1, dd1, v1, W_in[1], W_out[1], W_loop[1],
                         W_rel[1], loop_rel[1], conv_bias[1], alive1)

    xp2, alive2 = _pool_exact(x1, alive1, pool_p[1], k2)
    v2 = jnp.take(alive2, src) & jnp.take(alive2, dst)
    ds2, _, dd2_in, dd2_out = _level_degrees(idx_cat, dir_in, v2)
    gidx = src + off
    dstc1 = jnp.where(v1, dst + off, 2 * _N)
    dstc2 = jnp.where(v2, dst + off, 2 * _N)

    dsv = jnp.take(jnp.stack([ds2, ds1, ds0], axis=1), src + off, axis=0)
    vals3 = jnp.concatenate([
        jnp.where(v2, dsv[:, 0], 0.0),
        jnp.where(v1, dsv[:, 1], 0.0),
        dsv[:, 2],
    ])
    dst3 = jnp.concatenate([dst, dst, dst])
    col3 = jnp.concatenate([setp, setp + 4 * _R, setp + 8 * _R])
    S_all = jnp.zeros((_N, 12 * _R), jnp.float32).at[dst3, col3].add(vals3)

    x2, _ = _conv_fast(xp2, alive2, gidx, dstc2, S_all[:, :4 * _R],
                       ds2, dd2_in, dd2_out, r1,
                       W_in[2], W_out[2], W_loop[2], W_rel[2], loop_rel[2],
                       conv_bias[2])

    x3, _ = _conv_fast(x1 + x2, alive1, gidx, dstc1, S_all[:, 4 * _R:8 * _R],
                       ds1, dd1_in, dd1_out, r1,
                       W_in[3], W_out[3], W_loop[3], W_rel[3], loop_rel[3],
                       conv_bias[3])

    x4, r4 = _conv_fast(x0 + x3, ones_n, gidx, dst + off, S_all[:, 8 * _R:],
                        ds0, dd0_in, dd0_out, r0,
                        W_in[4], W_out[4], W_loop[4], W_rel[4], loop_rel[4],
                        conv_bias[4])

    sub_emb = jnp.take(x4, sub, axis=0)
    rel_emb = jnp.take(r4, rel, axis=0)
    return sub_emb, rel_emb, x4


# speedup vs baseline: 1.0423x; 1.0423x over previous
import functools
import math

import jax
import jax.numpy as jnp
from jax import lax
from jax.experimental import pallas as pl
from jax.experimental.pallas import tpu as pltpu
from jax.experimental.pallas import tpu_sc as plsc

_N = 10000
_R = 100
_E = 160000
_D = 128
_RATIO = 0.5

_NC = 2
_NS = 16
_EPT = _E // _NS
_CH = 80
_NCHUNK = _EPT // _CH
_ACC_ROWS = 10240


def _edge_body(table_hbm, gidx_hbm, dstc_hbm, zeros_hbm, out_hbm,
               gall, dall, sb0, sb1, rows0, rows1, acc, sem0, sem1):
    c = lax.axis_index("c")
    s = lax.axis_index("s")
    lo = c * _N
    hi = lo + _N
    base_e = s * _EPT

    pltpu.sync_copy(zeros_hbm.at[pl.ds(0, _ACC_ROWS // _NS)],
                    acc.at[pl.ds(s * (_ACC_ROWS // _NS), _ACC_ROWS // _NS)])
    pltpu.sync_copy(gidx_hbm.at[pl.ds(base_e, _EPT)], gall)
    pltpu.sync_copy(dstc_hbm.at[pl.ds(base_e, _EPT)], dall)
    plsc.subcore_barrier()

    trash = jnp.arange(16, dtype=jnp.int32) + _N

    def fire(ci, rbuf, sem):
        pltpu.async_copy(table_hbm.at[gall.at[pl.ds(ci * _CH, _CH)]],
                         rbuf, sem)

    def drain(rbuf, sem):
        pltpu.make_async_copy(zeros_hbm.at[pl.ds(0, _CH)], rbuf, sem).wait()

    def process(ci, rbuf, sbuf):
        for j in range(_CH // 16):
            dv = dall[pl.ds(ci * _CH + j * 16, 16)]
            mine = (dv >= lo) & (dv < hi)
            sbuf[pl.ds(j * 16, 16)] = jnp.where(mine, dv - lo, trash)
        pltpu.sync_copy(rbuf, acc.at[sbuf], add=True)

    fire(0, rows0, sem0)

    def lbody(i, carry):
        fire(2 * i + 1, rows1, sem1)
        drain(rows0, sem0)
        process(2 * i, rows0, sb0)
        fire(jnp.minimum(2 * i + 2, _NCHUNK - 1), rows0, sem0)
        drain(rows1, sem1)
        process(2 * i + 1, rows1, sb1)
        return carry

    lax.fori_loop(0, (_NCHUNK - 1) // 2, lbody, 0)
    drain(rows0, sem0)
    process(_NCHUNK - 1, rows0, sb0)
    plsc.subcore_barrier()

    @pl.when(s == 0)
    def _():
        pltpu.sync_copy(acc.at[pl.ds(0, _N)], out_hbm.at[pl.ds(lo, _N)])


def _sc_edge_pass(table, gidx, dstc):
    mesh = plsc.VectorSubcoreMesh(core_axis_name="c", subcore_axis_name="s",
                                  num_cores=_NC, num_subcores=_NS)
    f = pl.kernel(
        _edge_body,
        out_type=jax.ShapeDtypeStruct((2 * _N, _D), jnp.float32),
        mesh=mesh,
        scratch_types=[
            pltpu.VMEM((_EPT,), jnp.int32),
            pltpu.VMEM((_EPT,), jnp.int32),
            pltpu.VMEM((_CH,), jnp.int32),
            pltpu.VMEM((_CH,), jnp.int32),
            pltpu.VMEM((_CH, _D), jnp.float32),
            pltpu.VMEM((_CH, _D), jnp.float32),
            pltpu.VMEM_SHARED((_ACC_ROWS, _D), jnp.float32),
            pltpu.SemaphoreType.DMA,
            pltpu.SemaphoreType.DMA,
        ],
    )
    zeros = jnp.zeros((_ACC_ROWS // _NS, _D), jnp.float32)
    return f(table, gidx, dstc, zeros)


def _mm_body(x_ref, w_ref, o_ref):
    o_ref[...] = jnp.dot(x_ref[...], w_ref[...],
                         preferred_element_type=jnp.float32,
                         precision=jax.lax.Precision.HIGHEST)


def _mm(x, w, bm=2000):
    m, k = x.shape
    j = w.shape[1]
    if m % bm != 0:
        return pl.pallas_call(
            _mm_body,
            out_shape=jax.ShapeDtypeStruct((m, j), jnp.float32),
        )(x, w)
    return pl.pallas_call(
        _mm_body,
        grid=(m // bm,),
        in_specs=[pl.BlockSpec((bm, k), lambda i: (i, 0)),
                  pl.BlockSpec((k, j), lambda i: (0, 0))],
        out_specs=pl.BlockSpec((bm, j), lambda i: (i, 0)),
        out_shape=jax.ShapeDtypeStruct((m, j), jnp.float32),
    )(x, w)


def _fast_epilogue_body(ai_ref, ao_ref, p2_ref, dd_ref, lm_ref, b_ref,
                        alive_ref, o_ref):
    agg = (ai_ref[...] - p2_ref[:, :_D]) * dd_ref[:, 0:1] \
        + (ao_ref[...] - p2_ref[:, _D:]) * dd_ref[:, 1:2]
    val = jnp.tanh((agg + lm_ref[...]) / 3.0 + b_ref[...])
    o_ref[...] = val * alive_ref[...]


def _fast_epilogue(agg_in, agg_out, p2, di_d_in, di_d_out, loop_msg, bias,
                   alive):
    n, d = agg_in.shape
    dd = jnp.stack([di_d_in, di_d_out], axis=1)
    return pl.pallas_call(
        _fast_epilogue_body,
        out_shape=jax.ShapeDtypeStruct((n, d), jnp.float32),
    )(agg_in, agg_out, p2, dd, loop_msg,
      jnp.broadcast_to(bias[None, :], (n, d)),
      jnp.broadcast_to(alive[:, None].astype(jnp.float32), (n, d)))


def _di(deg):
    safe = jnp.where(deg > 0, deg, 1.0)
    return jnp.where(deg > 0, safe ** -0.5, 0.0)


def _level_degrees(idx_cat, dir_in, valid):
    w2 = jnp.stack([dir_in & valid, (~dir_in) & valid],
                   axis=1).astype(jnp.float32)
    w4 = jnp.concatenate([w2, w2], axis=0)
    deg = jnp.zeros((2 * _N, 2), jnp.float32).at[idx_cat].add(w4)
    di_d = _di(deg[:_N])
    di_s = _di(deg[_N:])
    di_s_cat = jnp.concatenate([di_s[:, 0], di_s[:, 1]])
    di_d_cat = jnp.concatenate([di_d[:, 0], di_d[:, 1]])
    return di_s_cat, di_d_cat, di_d[:, 0], di_d[:, 1]


def _conv_exact(x, rel_t, src, dst, et, dir_in, off, di_s_cat, di_d_cat,
                valid, W_in, W_out, W_loop, W_rel, loop_rel, bias, alive):
    comp = jnp.take(x, src, axis=0) - jnp.take(rel_t, et, axis=0)
    di_prod = jnp.take(di_s_cat, src + off) * jnp.take(di_d_cat, dst + off)
    ni = jnp.where(dir_in & valid, di_prod, 0.0)
    no = jnp.where((~dir_in) & valid, di_prod, 0.0)
    msg = (comp @ W_in) * ni[:, None] + (comp @ W_out) * no[:, None]
    agg = jnp.zeros((_N, _D), x.dtype).at[dst].add(msg)
    loop_msg = (x - loop_rel[None, :]) @ W_loop
    out = jnp.tanh((agg + loop_msg) / 3.0 + bias)
    return out * alive[:, None], rel_t @ W_rel


def _conv_fast(x, alive, gidx, dst_cat, S, di_s_cat, di_d_in,
               di_d_out, rel_t, W_in, W_out, W_loop, W_rel, loop_rel,
               bias):
    y = _mm(x, jnp.concatenate([W_in, W_out, W_loop], axis=1))
    xw_in, xw_out, xw_loop = y[:, :_D], y[:, _D:2 * _D], y[:, 2 * _D:]
    rw = _mm(rel_t, jnp.concatenate([W_in, W_out, W_rel], axis=1))
    rw_in, rw_out, rel_new = rw[:, :_D], rw[:, _D:2 * _D], rw[:, 2 * _D:]

    di_s_in, di_s_out = di_s_cat[:_N], di_s_cat[_N:]
    table = jnp.concatenate([
        xw_in * di_s_in[:, None],
        xw_out * di_s_out[:, None],
    ], axis=0)

    agg_cat = _sc_edge_pass(table, gidx, dst_cat)

    rw_blk = jnp.block([[rw_in, jnp.zeros((2 * _R, _D), jnp.float32)],
                        [jnp.zeros((2 * _R, _D), jnp.float32), rw_out]])
    p2 = _mm(S, rw_blk)

    loop_msg = xw_loop - (loop_rel @ W_loop)[None, :]
    x_new = _fast_epilogue(agg_cat[:_N], agg_cat[_N:], p2, di_d_in, di_d_out,
                           loop_msg, bias, alive)
    return x_new, rel_new


def _pool_exact(x, alive, p, k):
    score = jnp.tanh((x @ p) / jnp.linalg.norm(p))
    keys = jnp.where(alive, score, -jnp.inf)
    _, perm = jax.lax.top_k(keys, k)
    alive_new = jnp.zeros((_N,), bool).at[perm].set(True)
    x_new = x * score[:, None] * alive_new[:, None].astype(x.dtype)
    return x_new, alive_new


def kernel(sub, rel, edge_index, edge_type, init_embed, init_rel, W_in,
           W_out, W_loop, W_rel, loop_rel, conv_bias, pool_p):
    src, dst = edge_index[0], edge_index[1]
    et = edge_type

    ones_n = jnp.ones((_N,), bool)
    ones_e = jnp.ones((_E,), bool)

    k1 = int(math.ceil(_RATIO * _N))
    k2 = int(math.ceil(_RATIO * k1))

    dir_in = et < _R
    off = jnp.where(dir_in, 0, _N).astype(src.dtype)
    setp = (et + jnp.where(dir_in, 0, 2 * _R)).astype(src.dtype)

    idx_cat = jnp.concatenate([dst, src + _N])

    ds0, dd0, dd0_in, dd0_out = _level_degrees(idx_cat, dir_in, ones_e)
    x0, r0 = _conv_exact(init_embed, init_rel, src, dst, et, dir_in, off,
                         ds0, dd0, ones_e, W_in[0], W_out[0], W_loop[0],
                         W_rel[0], loop_rel[0], conv_bias[0], ones_n)

    xp1, alive1 = _pool_exact(x0, ones_n, pool_p[0], k1)
    v1 = jnp.take(alive1, src) & jnp.take(alive1, dst)
    ds1, dd1, dd1_in, dd1_out = _level_degrees(idx_cat, dir_in, v1)
    x1, r1 = _conv_exact(xp1, r0, src, dst, et, dir_in, off,
                         ds1, dd1, v1, W_in[1], W_out[1], W_loop[1],
                         W_rel[1], loop_rel[1], conv_bias[1], alive1)

    xp2, alive2 = _pool_exact(x1, alive1, pool_p[1], k2)
    v2 = jnp.take(alive2, src) & jnp.take(alive2, dst)
    ds2, _, dd2_in, dd2_out = _level_degrees(idx_cat, dir_in, v2)
    gidx = src + off
    dstc1 = jnp.where(v1, dst + off, 2 * _N)
    dstc2 = jnp.where(v2, dst + off, 2 * _N)

    dsv = jnp.take(jnp.stack([ds2, ds1, ds0], axis=1), src + off, axis=0)

    def _smat(vals):
        return jnp.zeros((_N, 4 * _R), jnp.float32).at[dst, setp].add(vals)

    S2 = _smat(jnp.where(v2, dsv[:, 0], 0.0))
    S1 = _smat(jnp.where(v1, dsv[:, 1], 0.0))
    S0 = _smat(dsv[:, 2])

    x2, _ = _conv_fast(xp2, alive2, gidx, dstc2, S2,
                       ds2, dd2_in, dd2_out, r1,
                       W_in[2], W_out[2], W_loop[2], W_rel[2], loop_rel[2],
                       conv_bias[2])

    x3, _ = _conv_fast(x1 + x2, alive1, gidx, dstc1, S1,
                       ds1, dd1_in, dd1_out, r1,
                       W_in[3], W_out[3], W_loop[3], W_rel[3], loop_rel[3],
                       conv_bias[3])

    x4, r4 = _conv_fast(x0 + x3, ones_n, gidx, dst + off, S0,
                        ds0, dd0_in, dd0_out, r0,
                        W_in[4], W_out[4], W_loop[4], W_rel[4], loop_rel[4],
                        conv_bias[4])

    sub_emb = jnp.take(x4, sub, axis=0)
    rel_emb = jnp.take(r4, rel, axis=0)
    return sub_emb, rel_emb, x4
